# baseline (speedup 1.0000x reference)
import functools

import jax
import jax.numpy as jnp
from jax import lax
from jax.experimental import pallas as pl
from jax.experimental.pallas import tpu as pltpu
from jax.experimental.pallas import tpu_sc as plsc

NC = 2
NS = 16
NW = NC * NS
EB = 128
NBUF = 3
DCH = 8


def _mesh():
  return plsc.VectorSubcoreMesh(core_axis_name="c", subcore_axis_name="s")


def _make_degree_kernel(npad, bpw):
  gpw = bpw // DCH

  @functools.partial(
      pl.kernel,
      out_type=jax.ShapeDtypeStruct((NC * npad,), jnp.float32),
      mesh=_mesh(),
      scratch_types=(
          [pltpu.VMEM((DCH, EB), jnp.int32)] * 2
          + [pltpu.VMEM((EB,), jnp.float32)]
          + [pltpu.VMEM_SHARED((npad,), jnp.float32)]
          + [pltpu.SemaphoreType.DMA] * 2
      ),
  )
  def deg_kernel(dstm_hbm, zeros_hbm, out_hbm, *scr):
    didx = list(scr[:2])
    ones_v = scr[2]
    acc_sh = scr[3]
    sems = list(scr[4:])

    c = lax.axis_index("c")
    s = lax.axis_index("s")
    wid = c * NS + s
    gbase = wid * gpw

    for i in range(EB // 16):
      ones_v[pl.ds(i * 16, 16)] = jnp.ones((16,), jnp.float32)

    @pl.when(s == 0)
    def _():
      pltpu.sync_copy(zeros_hbm, acc_sh)
    plsc.subcore_barrier()

    def fire(p):
      for k in range(DCH):
        pltpu.async_copy(ones_v, acc_sh.at[didx[p].at[k]], sems[p], add=True)

    def drain(p):
      for k in range(DCH):
        pltpu.make_async_copy(ones_v, acc_sh.at[didx[p].at[k]], sems[p]).wait()

    for p in range(2):
      pltpu.sync_copy(dstm_hbm.at[pl.ds((gbase + p) * DCH, DCH)], didx[p])
      fire(p)

    def body(t2, carry):
      for p in range(2):
        g = 2 * t2 + 2 + p
        drain(p)
        pltpu.sync_copy(dstm_hbm.at[pl.ds((gbase + g) * DCH, DCH)], didx[p])
        fire(p)
      return carry

    lax.fori_loop(0, (gpw - 2) // 2, body, 0)
    drain(0)
    drain(1)
    plsc.subcore_barrier()

    @pl.when(s == 0)
    def _():
      pltpu.sync_copy(acc_sh, out_hbm.at[pl.ds(c * npad, npad)])

  return deg_kernel


def _make_agg_kernel(npad, d, bpw):
  outer = bpw // NBUF
  rows_per_tile = npad // NS

  @functools.partial(
      pl.kernel,
      out_type=jax.ShapeDtypeStruct((NC * npad, d), jnp.float32),
      mesh=_mesh(),
      scratch_types=(
          [pltpu.VMEM((2, EB), jnp.int32)] * NBUF
          + [pltpu.VMEM((EB, d), jnp.float32)] * NBUF
          + [pltpu.VMEM_SHARED((npad, d), jnp.float32)]
          + [pltpu.SemaphoreType.DMA] * (2 * NBUF)
      ),
  )
  def agg_kernel(xt_hbm, idx_hbm, zeros_hbm, out_hbm, *scr):
    idx = list(scr[:NBUF])
    rows = list(scr[NBUF:2 * NBUF])
    acc_sh = scr[2 * NBUF]
    semg = list(scr[2 * NBUF + 1:2 * NBUF + 1 + NBUF])
    sems = list(scr[2 * NBUF + 1 + NBUF:])

    c = lax.axis_index("c")
    s = lax.axis_index("s")
    wid = c * NS + s
    bbase = wid * bpw

    r0 = s * rows_per_tile
    pltpu.sync_copy(zeros_hbm.at[pl.ds(r0, rows_per_tile)],
                    acc_sh.at[pl.ds(r0, rows_per_tile)])
    plsc.subcore_barrier()

    for j in range(NBUF):
      pltpu.sync_copy(idx_hbm.at[bbase + j], idx[j])
      pltpu.async_copy(xt_hbm.at[idx[j].at[0]], rows[j], semg[j])

    def body(t, carry):
      for j in range(NBUF):
        pltpu.make_async_copy(xt_hbm.at[idx[j].at[0]], rows[j], semg[j]).wait()
        pltpu.async_copy(rows[j], acc_sh.at[idx[j].at[1]], sems[j], add=True)
      for j in range(NBUF):
        b = t * NBUF + j
        pltpu.make_async_copy(rows[j], acc_sh.at[idx[j].at[1]], sems[j]).wait()
        pltpu.sync_copy(idx_hbm.at[bbase + b + NBUF], idx[j])
        pltpu.async_copy(xt_hbm.at[idx[j].at[0]], rows[j], semg[j])
      return carry

    lax.fori_loop(0, outer - 1, body, 0)

    for j in range(NBUF):
      pltpu.make_async_copy(xt_hbm.at[idx[j].at[0]], rows[j], semg[j]).wait()
      pltpu.async_copy(rows[j], acc_sh.at[idx[j].at[1]], sems[j], add=True)
    for j in range(NBUF):
      pltpu.make_async_copy(rows[j], acc_sh.at[idx[j].at[1]], sems[j]).wait()
    plsc.subcore_barrier()

    pltpu.sync_copy(acc_sh.at[pl.ds(r0, rows_per_tile)],
                    out_hbm.at[pl.ds(c * npad + r0, rows_per_tile)])

  return agg_kernel


def _scale_body(x_ref, d0_ref, d1_ref, o_ref):
  cnt = d0_ref[0] + d1_ref[0] + 1.0
  dis = lax.rsqrt(cnt)
  o_ref[...] = x_ref[...] * dis


def _combine_body(ya_ref, yb_ref, x_ref, d0_ref, d1_ref,
                  wc_ref, wl_ref, bc_ref, bl_ref, o_ref):
  cnt = d0_ref[0] + d1_ref[0] + 1.0
  dis = lax.rsqrt(cnt)
  y = dis * (ya_ref[0] + yb_ref[0]) + (dis * dis) * x_ref[...]
  dn = (((1,), (1,)), ((), ()))
  agg = lax.dot_general(y, wc_ref[...], dn,
                        preferred_element_type=jnp.float32) + bc_ref[...]
  o_ref[...] = lax.dot_general(agg, wl_ref[...], dn,
                               preferred_element_type=jnp.float32) + bl_ref[...]


def _pick_grid(n):
  for nb in (25, 20, 16, 10, 8, 5, 4, 2, 1):
    if n % nb == 0 and (n // nb) % 8 == 0:
      return nb
  return 1


def kernel(x, edge_index, W_conv, b_conv, W_lin, b_lin):
  n, d = x.shape
  e = edge_index.shape[1]

  npad = ((n + 127) // 128) * 128
  unit = 2 * NBUF
  bpw = ((e + NW * EB - 1) // (NW * EB) + unit - 1) // unit * unit
  bpw_deg = (bpw // (2 * DCH)) * (2 * DCH)
  if bpw_deg * NW * EB < e:
    bpw_deg += 2 * DCH
    bpw = max(bpw, ((bpw_deg + unit - 1) // unit) * unit)
  e2 = bpw * NW * EB

  src = edge_index[0]
  dst = edge_index[1]
  pad_r = jnp.arange(e2 - e, dtype=jnp.int32)
  src_p = jnp.concatenate([src, jnp.zeros((e2 - e,), jnp.int32)])
  dst_p = jnp.concatenate([dst, n + pad_r % (npad - n)])
  src_m = src_p.reshape(-1, EB)
  dst_m = dst_p.reshape(-1, EB)
  idx2 = jnp.stack([src_m, dst_m], axis=1)
  zeros_1d = jnp.zeros((npad,), jnp.float32)
  zeros_2d = jnp.zeros((npad, d), jnp.float32)

  degp = _make_degree_kernel(npad, bpw_deg)(dst_m, zeros_1d)
  deg3 = degp.reshape(2, npad, 1)

  nb = _pick_grid(n)
  br = n // nb
  dspec0 = pl.BlockSpec((1, br, 1), lambda i: (0, i, 0))
  dspec1 = pl.BlockSpec((1, br, 1), lambda i: (1, i, 0))
  xt = pl.pallas_call(
      _scale_body,
      grid=(nb,),
      in_specs=[pl.BlockSpec((br, d), lambda i: (i, 0)), dspec0, dspec1],
      out_specs=pl.BlockSpec((br, d), lambda i: (i, 0)),
      out_shape=jax.ShapeDtypeStruct((n, d), jnp.float32),
  )(x, deg3, deg3)

  y0 = _make_agg_kernel(npad, d, bpw)(xt, idx2, zeros_2d)
  y3 = y0.reshape(2, npad, d)

  h = W_conv.shape[0]
  out = pl.pallas_call(
      _combine_body,
      grid=(nb,),
      in_specs=[
          pl.BlockSpec((1, br, d), lambda i: (0, i, 0)),
          pl.BlockSpec((1, br, d), lambda i: (1, i, 0)),
          pl.BlockSpec((br, d), lambda i: (i, 0)),
          dspec0,
          dspec1,
          pl.BlockSpec((h, d), lambda i: (0, 0)),
          pl.BlockSpec((d, h), lambda i: (0, 0)),
          pl.BlockSpec((1, h), lambda i: (0, 0)),
          pl.BlockSpec((1, d), lambda i: (0, 0)),
      ],
      out_specs=pl.BlockSpec((br, d), lambda i: (i, 0)),
      out_shape=jax.ShapeDtypeStruct((n, d), jnp.float32),
  )(y3, y3, x, deg3, deg3,
    W_conv, W_lin, b_conv.reshape(1, h), b_lin.reshape(1, d))

  return out


# speedup vs baseline: 6.3274x; 6.3274x over previous
import functools

import jax
import jax.numpy as jnp
from jax import lax
from jax.experimental import pallas as pl
from jax.experimental.pallas import tpu as pltpu
from jax.experimental.pallas import tpu_sc as plsc

NC = 2
NS = 16
NW = NC * NS
EB = 128
NBUF = 3
DCH = 8


def _mesh():
  return plsc.VectorSubcoreMesh(core_axis_name="c", subcore_axis_name="s")


def _make_degree_kernel(npad, bpw):
  gpw = bpw // DCH

  @functools.partial(
      pl.kernel,
      out_type=jax.ShapeDtypeStruct((NC * npad,), jnp.float32),
      mesh=_mesh(),
      scratch_types=(
          [pltpu.VMEM((DCH, EB), jnp.int32)] * 2
          + [pltpu.VMEM((EB,), jnp.float32)]
          + [pltpu.VMEM_SHARED((npad,), jnp.float32)]
          + [pltpu.SemaphoreType.DMA] * 2
      ),
  )
  def deg_kernel(dstm_hbm, zeros_hbm, out_hbm, *scr):
    didx = list(scr[:2])
    ones_v = scr[2]
    acc_sh = scr[3]
    sems = list(scr[4:])

    c = lax.axis_index("c")
    s = lax.axis_index("s")
    wid = c * NS + s

    for i in range(EB // 16):
      ones_v[pl.ds(i * 16, 16)] = jnp.ones((16,), jnp.float32)

    @pl.when(s == 0)
    def _():
      pltpu.sync_copy(zeros_hbm, acc_sh)
    plsc.subcore_barrier()

    def fire(p):
      for k in range(DCH):
        pltpu.async_copy(ones_v, acc_sh.at[didx[p].at[k]], sems[p], add=True)

    def drain(p):
      for k in range(DCH):
        pltpu.make_async_copy(ones_v, acc_sh.at[didx[p].at[k]], sems[p]).wait()

    for p in range(2):
      off = pl.multiple_of((p * NW + wid) * DCH, DCH)
      pltpu.sync_copy(dstm_hbm.at[pl.ds(off, DCH)], didx[p])
      fire(p)

    def body(t2, carry):
      for p in range(2):
        g = 2 * t2 + 2 + p
        drain(p)
        off = pl.multiple_of((g * NW + wid) * DCH, DCH)
        pltpu.sync_copy(dstm_hbm.at[pl.ds(off, DCH)], didx[p])
        fire(p)
      return carry

    lax.fori_loop(0, (gpw - 2) // 2, body, 0)
    drain(0)
    drain(1)
    plsc.subcore_barrier()

    @pl.when(s == 0)
    def _():
      pltpu.sync_copy(acc_sh, out_hbm.at[pl.ds(c * npad, npad)])

  return deg_kernel


def _make_agg_kernel(npad, d, bpw):
  outer = bpw // NBUF
  rows_per_tile = npad // NS

  @functools.partial(
      pl.kernel,
      out_type=jax.ShapeDtypeStruct((NC * npad, d), jnp.float32),
      mesh=_mesh(),
      scratch_types=(
          [pltpu.VMEM((2, EB), jnp.int32)] * NBUF
          + [pltpu.VMEM((EB, d), jnp.float32)] * NBUF
          + [pltpu.VMEM_SHARED((npad, d), jnp.float32)]
          + [pltpu.SemaphoreType.DMA] * (2 * NBUF)
      ),
  )
  def agg_kernel(xt_hbm, idx_hbm, zeros_hbm, out_hbm, *scr):
    idx = list(scr[:NBUF])
    rows = list(scr[NBUF:2 * NBUF])
    acc_sh = scr[2 * NBUF]
    semg = list(scr[2 * NBUF + 1:2 * NBUF + 1 + NBUF])
    sems = list(scr[2 * NBUF + 1 + NBUF:])

    c = lax.axis_index("c")
    s = lax.axis_index("s")
    wid = c * NS + s

    r0 = s * rows_per_tile
    pltpu.sync_copy(zeros_hbm.at[pl.ds(r0, rows_per_tile)],
                    acc_sh.at[pl.ds(r0, rows_per_tile)])
    plsc.subcore_barrier()

    for j in range(NBUF):
      pltpu.sync_copy(idx_hbm.at[j * NW + wid], idx[j])
      pltpu.async_copy(xt_hbm.at[idx[j].at[0]], rows[j], semg[j])

    def body(t, carry):
      for j in range(NBUF):
        pltpu.make_async_copy(xt_hbm.at[idx[j].at[0]], rows[j], semg[j]).wait()
        pltpu.async_copy(rows[j], acc_sh.at[idx[j].at[1]], sems[j], add=True)
      for j in range(NBUF):
        b = t * NBUF + j
        pltpu.make_async_copy(rows[j], acc_sh.at[idx[j].at[1]], sems[j]).wait()
        pltpu.sync_copy(idx_hbm.at[(b + NBUF) * NW + wid], idx[j])
        pltpu.async_copy(xt_hbm.at[idx[j].at[0]], rows[j], semg[j])
      return carry

    lax.fori_loop(0, outer - 1, body, 0)

    for j in range(NBUF):
      pltpu.make_async_copy(xt_hbm.at[idx[j].at[0]], rows[j], semg[j]).wait()
      pltpu.async_copy(rows[j], acc_sh.at[idx[j].at[1]], sems[j], add=True)
    for j in range(NBUF):
      pltpu.make_async_copy(rows[j], acc_sh.at[idx[j].at[1]], sems[j]).wait()
    plsc.subcore_barrier()

    pltpu.sync_copy(acc_sh.at[pl.ds(r0, rows_per_tile)],
                    out_hbm.at[pl.ds(c * npad + r0, rows_per_tile)])

  return agg_kernel


def _scale_body(x_ref, d0_ref, d1_ref, o_ref):
  cnt = d0_ref[0] + d1_ref[0] + 1.0
  dis = lax.rsqrt(cnt)
  o_ref[...] = x_ref[...] * dis


def _combine_body(ya_ref, yb_ref, x_ref, d0_ref, d1_ref,
                  wc_ref, wl_ref, bc_ref, bl_ref, o_ref):
  cnt = d0_ref[0] + d1_ref[0] + 1.0
  dis = lax.rsqrt(cnt)
  y = dis * (ya_ref[0] + yb_ref[0]) + (dis * dis) * x_ref[...]
  dn = (((1,), (1,)), ((), ()))
  agg = lax.dot_general(y, wc_ref[...], dn,
                        preferred_element_type=jnp.float32) + bc_ref[...]
  o_ref[...] = lax.dot_general(agg, wl_ref[...], dn,
                               preferred_element_type=jnp.float32) + bl_ref[...]


def _pick_grid(n):
  for nb in (25, 20, 16, 10, 8, 5, 4, 2, 1):
    if n % nb == 0 and (n // nb) % 8 == 0:
      return nb
  return 1


def kernel(x, edge_index, W_conv, b_conv, W_lin, b_lin):
  n, d = x.shape
  e = edge_index.shape[1]

  npad = ((n + 127) // 128) * 128
  unit = NBUF
  bpw = ((e + NW * EB - 1) // (NW * EB) + unit - 1) // unit * unit
  bpw_deg = (bpw // (2 * DCH)) * (2 * DCH)
  if bpw_deg * NW * EB < e:
    bpw_deg += 2 * DCH
    bpw = max(bpw, ((bpw_deg + unit - 1) // unit) * unit)
  e2 = bpw * NW * EB

  src = edge_index[0]
  dst = edge_index[1]
  pad_r = jnp.arange(e2 - e, dtype=jnp.int32)
  src_p = jnp.concatenate([src, pad_r % n])
  dst_p = jnp.concatenate([dst, n + pad_r % (npad - n)])
  src_m = src_p.reshape(-1, EB)
  dst_m = dst_p.reshape(-1, EB)
  idx2 = jnp.stack([src_m, dst_m], axis=1)
  zeros_1d = jnp.zeros((npad,), jnp.float32)
  zeros_2d = jnp.zeros((npad, d), jnp.float32)

  degp = _make_degree_kernel(npad, bpw_deg)(dst_m, zeros_1d)
  deg3 = degp.reshape(2, npad, 1)

  nb = _pick_grid(n)
  br = n // nb
  dspec0 = pl.BlockSpec((1, br, 1), lambda i: (0, i, 0))
  dspec1 = pl.BlockSpec((1, br, 1), lambda i: (1, i, 0))
  xt = pl.pallas_call(
      _scale_body,
      grid=(nb,),
      in_specs=[pl.BlockSpec((br, d), lambda i: (i, 0)), dspec0, dspec1],
      out_specs=pl.BlockSpec((br, d), lambda i: (i, 0)),
      out_shape=jax.ShapeDtypeStruct((n, d), jnp.float32),
  )(x, deg3, deg3)

  y0 = _make_agg_kernel(npad, d, bpw)(xt, idx2, zeros_2d)
  y3 = y0.reshape(2, npad, d)

  h = W_conv.shape[0]
  out = pl.pallas_call(
      _combine_body,
      grid=(nb,),
      in_specs=[
          pl.BlockSpec((1, br, d), lambda i: (0, i, 0)),
          pl.BlockSpec((1, br, d), lambda i: (1, i, 0)),
          pl.BlockSpec((br, d), lambda i: (i, 0)),
          dspec0,
          dspec1,
          pl.BlockSpec((h, d), lambda i: (0, 0)),
          pl.BlockSpec((d, h), lambda i: (0, 0)),
          pl.BlockSpec((1, h), lambda i: (0, 0)),
          pl.BlockSpec((1, d), lambda i: (0, 0)),
      ],
      out_specs=pl.BlockSpec((br, d), lambda i: (i, 0)),
      out_shape=jax.ShapeDtypeStruct((n, d), jnp.float32),
  )(y3, y3, x, deg3, deg3,
    W_conv, W_lin, b_conv.reshape(1, h), b_lin.reshape(1, d))

  return out


# speedup vs baseline: 6.8493x; 1.0825x over previous
import functools

import jax
import jax.numpy as jnp
from jax import lax
from jax.experimental import pallas as pl
from jax.experimental.pallas import tpu as pltpu
from jax.experimental.pallas import tpu_sc as plsc

NC = 2
NS = 16
NW = NC * NS
EB = 128
NBUF = 3
DCH = 8


def _mesh():
  return plsc.VectorSubcoreMesh(core_axis_name="c", subcore_axis_name="s")


def _make_degree_kernel(npad, bpw):
  gpw = bpw // DCH

  @functools.partial(
      pl.kernel,
      out_type=jax.ShapeDtypeStruct((NC * npad,), jnp.float32),
      mesh=_mesh(),
      scratch_types=(
          [pltpu.VMEM((DCH, EB), jnp.int32)] * 2
          + [pltpu.VMEM((EB,), jnp.float32)]
          + [pltpu.VMEM_SHARED((npad,), jnp.float32)]
          + [pltpu.SemaphoreType.DMA] * 2
      ),
  )
  def deg_kernel(edge_hbm, zeros_hbm, out_hbm, *scr):
    didx = list(scr[:2])
    ones_v = scr[2]
    acc_sh = scr[3]
    sems = list(scr[4:])

    c = lax.axis_index("c")
    s = lax.axis_index("s")
    wid = c * NS + s

    for i in range(EB // 16):
      ones_v[pl.ds(i * 16, 16)] = jnp.ones((16,), jnp.float32)

    @pl.when(s == 0)
    def _():
      pltpu.sync_copy(zeros_hbm, acc_sh)
    plsc.subcore_barrier()

    def fire(p):
      for k in range(DCH):
        pltpu.async_copy(ones_v, acc_sh.at[didx[p].at[k]], sems[p], add=True)

    def drain(p):
      for k in range(DCH):
        pltpu.make_async_copy(ones_v, acc_sh.at[didx[p].at[k]], sems[p]).wait()

    for p in range(2):
      off = pl.multiple_of((p * NW + wid) * DCH, DCH)
      pltpu.sync_copy(edge_hbm.at[1, pl.ds(off, DCH)], didx[p])
      fire(p)

    def body(t2, carry):
      for p in range(2):
        g = 2 * t2 + 2 + p
        drain(p)
        off = pl.multiple_of((g * NW + wid) * DCH, DCH)
        pltpu.sync_copy(edge_hbm.at[1, pl.ds(off, DCH)], didx[p])
        fire(p)
      return carry

    lax.fori_loop(0, (gpw - 2) // 2, body, 0)
    drain(0)
    drain(1)
    plsc.subcore_barrier()

    @pl.when(s == 0)
    def _():
      pltpu.sync_copy(acc_sh, out_hbm.at[pl.ds(c * npad, npad)])

  return deg_kernel


def _make_agg_kernel(npad, d, bpw):
  ngrp = bpw // NBUF
  rows_per_tile = npad // NS

  @functools.partial(
      pl.kernel,
      out_type=jax.ShapeDtypeStruct((NC * npad, d), jnp.float32),
      mesh=_mesh(),
      scratch_types=(
          [pltpu.VMEM((EB,), jnp.int32)] * NBUF
          + [pltpu.VMEM((EB,), jnp.int32)] * NBUF
          + [pltpu.VMEM((EB, d), jnp.float32)] * NBUF
          + [pltpu.VMEM_SHARED((npad, d), jnp.float32)]
          + [pltpu.SemaphoreType.DMA] * (2 * NBUF + 2)
      ),
  )
  def agg_kernel(xt_hbm, edge_hbm, zeros_hbm, out_hbm, *scr):
    sidx = list(scr[:NBUF])
    didx = list(scr[NBUF:2 * NBUF])
    rows = list(scr[2 * NBUF:3 * NBUF])
    acc_sh = scr[3 * NBUF]
    semg = list(scr[3 * NBUF + 1:4 * NBUF + 1])
    sems = list(scr[4 * NBUF + 1:5 * NBUF + 1])
    semis = scr[5 * NBUF + 1]
    semid = scr[5 * NBUF + 2]

    c = lax.axis_index("c")
    s = lax.axis_index("s")
    wid = c * NS + s

    r0 = s * rows_per_tile
    pltpu.sync_copy(zeros_hbm.at[pl.ds(r0, rows_per_tile)],
                    acc_sh.at[pl.ds(r0, rows_per_tile)])
    plsc.subcore_barrier()

    def gb_of(g, j):
      return (g * NBUF + j) * NW + wid

    for j in range(NBUF):
      pltpu.sync_copy(edge_hbm.at[0, gb_of(0, j)], sidx[j])
      pltpu.async_copy(edge_hbm.at[1, gb_of(0, j)], didx[j], semid)
    for j in range(NBUF):
      pltpu.async_copy(xt_hbm.at[sidx[j]], rows[j], semg[j])

    def body(g, carry):
      for j in range(NBUF):
        pltpu.make_async_copy(edge_hbm.at[1, wid], didx[j], semid).wait()
      for j in range(NBUF):
        pltpu.make_async_copy(xt_hbm.at[sidx[j]], rows[j], semg[j]).wait()
        pltpu.async_copy(rows[j], acc_sh.at[didx[j]], sems[j], add=True)
        pltpu.async_copy(edge_hbm.at[0, gb_of(g + 1, j)], sidx[j], semis)
      for j in range(NBUF):
        pltpu.make_async_copy(edge_hbm.at[0, wid], sidx[j], semis).wait()
      for j in range(NBUF):
        pltpu.make_async_copy(rows[j], acc_sh.at[didx[j]], sems[j]).wait()
        pltpu.async_copy(xt_hbm.at[sidx[j]], rows[j], semg[j])
        pltpu.async_copy(edge_hbm.at[1, gb_of(g + 1, j)], didx[j], semid)
      return carry

    lax.fori_loop(0, ngrp - 1, body, 0)

    for j in range(NBUF):
      pltpu.make_async_copy(edge_hbm.at[1, wid], didx[j], semid).wait()
    for j in range(NBUF):
      pltpu.make_async_copy(xt_hbm.at[sidx[j]], rows[j], semg[j]).wait()
      pltpu.async_copy(rows[j], acc_sh.at[didx[j]], sems[j], add=True)
    for j in range(NBUF):
      pltpu.make_async_copy(rows[j], acc_sh.at[didx[j]], sems[j]).wait()
    plsc.subcore_barrier()

    pltpu.sync_copy(acc_sh.at[pl.ds(r0, rows_per_tile)],
                    out_hbm.at[pl.ds(c * npad + r0, rows_per_tile)])

  return agg_kernel


def _scale_body(x_ref, d0_ref, d1_ref, o_ref):
  cnt = d0_ref[0] + d1_ref[0] + 1.0
  dis = lax.rsqrt(cnt)
  o_ref[...] = x_ref[...] * dis


def _combine_body(ya_ref, yb_ref, x_ref, d0_ref, d1_ref,
                  wc_ref, wl_ref, bc_ref, bl_ref, o_ref):
  cnt = d0_ref[0] + d1_ref[0] + 1.0
  dis = lax.rsqrt(cnt)
  y = dis * (ya_ref[0] + yb_ref[0]) + (dis * dis) * x_ref[...]
  dn = (((1,), (1,)), ((), ()))
  agg = lax.dot_general(y, wc_ref[...], dn,
                        preferred_element_type=jnp.float32) + bc_ref[...]
  o_ref[...] = lax.dot_general(agg, wl_ref[...], dn,
                               preferred_element_type=jnp.float32) + bl_ref[...]


def _pick_grid(n):
  for nb in (5, 8, 10, 4, 20, 16, 25, 2, 1):
    if n % nb == 0 and (n // nb) % 8 == 0:
      return nb
  return 1


def kernel(x, edge_index, W_conv, b_conv, W_lin, b_lin):
  n, d = x.shape
  e = edge_index.shape[1]

  npad = ((n + 127) // 128) * 128
  unit = NBUF
  bpw = ((e + NW * EB - 1) // (NW * EB) + unit - 1) // unit * unit
  bpw_deg = (bpw // (2 * DCH)) * (2 * DCH)
  if bpw_deg * NW * EB < e:
    bpw_deg += 2 * DCH
    bpw = max(bpw, ((bpw_deg + unit - 1) // unit) * unit)
  e2 = bpw * NW * EB
  nbatch = e2 // EB

  pad_r = jnp.arange(e2 - e, dtype=jnp.int32)
  pads = jnp.stack([pad_r % n, n + pad_r % (npad - n)])
  edge3 = jnp.concatenate([edge_index, pads], axis=1).reshape(2, nbatch, EB)
  zeros_1d = jnp.zeros((npad,), jnp.float32)
  zeros_2d = jnp.zeros((npad, d), jnp.float32)

  degp = _make_degree_kernel(npad, bpw_deg)(edge3, zeros_1d)
  deg3 = degp.reshape(2, npad, 1)

  nb = _pick_grid(n)
  br = n // nb
  dspec0 = pl.BlockSpec((1, br, 1), lambda i: (0, i, 0))
  dspec1 = pl.BlockSpec((1, br, 1), lambda i: (1, i, 0))
  xt = pl.pallas_call(
      _scale_body,
      grid=(nb,),
      in_specs=[pl.BlockSpec((br, d), lambda i: (i, 0)), dspec0, dspec1],
      out_specs=pl.BlockSpec((br, d), lambda i: (i, 0)),
      out_shape=jax.ShapeDtypeStruct((n, d), jnp.float32),
  )(x, deg3, deg3)

  y0 = _make_agg_kernel(npad, d, bpw)(xt, edge3, zeros_2d)
  y3 = y0.reshape(2, npad, d)

  h = W_conv.shape[0]
  out = pl.pallas_call(
      _combine_body,
      grid=(nb,),
      in_specs=[
          pl.BlockSpec((1, br, d), lambda i: (0, i, 0)),
          pl.BlockSpec((1, br, d), lambda i: (1, i, 0)),
          pl.BlockSpec((br, d), lambda i: (i, 0)),
          dspec0,
          dspec1,
          pl.BlockSpec((h, d), lambda i: (0, 0)),
          pl.BlockSpec((d, h), lambda i: (0, 0)),
          pl.BlockSpec((1, h), lambda i: (0, 0)),
          pl.BlockSpec((1, d), lambda i: (0, 0)),
      ],
      out_specs=pl.BlockSpec((br, d), lambda i: (i, 0)),
      out_shape=jax.ShapeDtypeStruct((n, d), jnp.float32),
  )(y3, y3, x, deg3, deg3,
    W_conv, W_lin, b_conv.reshape(1, h), b_lin.reshape(1, d))

  return out


# speedup vs baseline: 7.5007x; 1.0951x over previous
import functools

import jax
import jax.numpy as jnp
from jax import lax
from jax.experimental import pallas as pl
from jax.experimental.pallas import tpu as pltpu
from jax.experimental.pallas import tpu_sc as plsc

NC = 2
NS = 16
NW = NC * NS
EB = 128
NBUF = 3
DCH = 8


def _mesh():
  return plsc.VectorSubcoreMesh(core_axis_name="c", subcore_axis_name="s")


def _make_degree_kernel(npad, bpw):
  gpw = bpw // DCH

  @functools.partial(
      pl.kernel,
      out_type=jax.ShapeDtypeStruct((NC * npad,), jnp.float32),
      mesh=_mesh(),
      scratch_types=(
          [pltpu.VMEM((DCH, EB), jnp.int32)] * 2
          + [pltpu.VMEM((EB,), jnp.float32)]
          + [pltpu.VMEM_SHARED((npad,), jnp.float32)]
          + [pltpu.SemaphoreType.DMA] * 2
      ),
  )
  def deg_kernel(edge_hbm, zeros_hbm, out_hbm, *scr):
    didx = list(scr[:2])
    ones_v = scr[2]
    acc_sh = scr[3]
    sems = list(scr[4:])

    c = lax.axis_index("c")
    s = lax.axis_index("s")
    wid = c * NS + s

    for i in range(EB // 16):
      ones_v[pl.ds(i * 16, 16)] = jnp.ones((16,), jnp.float32)

    @pl.when(s == 0)
    def _():
      pltpu.sync_copy(zeros_hbm, acc_sh)
    plsc.subcore_barrier()

    def fire(p):
      for k in range(DCH):
        pltpu.async_copy(ones_v, acc_sh.at[didx[p].at[k]], sems[p], add=True)

    def drain(p):
      for k in range(DCH):
        pltpu.make_async_copy(ones_v, acc_sh.at[didx[p].at[k]], sems[p]).wait()

    for p in range(2):
      off = pl.multiple_of((p * NW + wid) * DCH, DCH)
      pltpu.sync_copy(edge_hbm.at[1, pl.ds(off, DCH)], didx[p])
      fire(p)

    def body(t2, carry):
      for p in range(2):
        g = 2 * t2 + 2 + p
        drain(p)
        off = pl.multiple_of((g * NW + wid) * DCH, DCH)
        pltpu.sync_copy(edge_hbm.at[1, pl.ds(off, DCH)], didx[p])
        fire(p)
      return carry

    lax.fori_loop(0, (gpw - 2) // 2, body, 0)
    drain(0)
    drain(1)
    plsc.subcore_barrier()

    @pl.when(s == 0)
    def _():
      pltpu.sync_copy(acc_sh, out_hbm.at[pl.ds(c * npad, npad)])

  return deg_kernel


def _make_agg_kernel(npad, d, bpw):
  ngrp = bpw // NBUF
  rows_per_tile = npad // NS

  @functools.partial(
      pl.kernel,
      out_type=jax.ShapeDtypeStruct((NC * npad, d), jnp.float32),
      mesh=_mesh(),
      scratch_types=(
          [pltpu.VMEM((EB,), jnp.int32)] * NBUF
          + [pltpu.VMEM((EB,), jnp.int32)] * NBUF
          + [pltpu.VMEM((EB, d), jnp.float32)] * NBUF
          + [pltpu.VMEM_SHARED((npad, d), jnp.float32)]
          + [pltpu.SemaphoreType.DMA] * (2 * NBUF + 2)
      ),
  )
  def agg_kernel(xt_hbm, edge_hbm, zeros_hbm, out_hbm, *scr):
    sidx = list(scr[:NBUF])
    didx = list(scr[NBUF:2 * NBUF])
    rows = list(scr[2 * NBUF:3 * NBUF])
    acc_sh = scr[3 * NBUF]
    semg = list(scr[3 * NBUF + 1:4 * NBUF + 1])
    sems = list(scr[4 * NBUF + 1:5 * NBUF + 1])
    semis = scr[5 * NBUF + 1]
    semid = scr[5 * NBUF + 2]

    c = lax.axis_index("c")
    s = lax.axis_index("s")
    wid = c * NS + s

    r0 = s * rows_per_tile
    pltpu.sync_copy(zeros_hbm.at[pl.ds(r0, rows_per_tile)],
                    acc_sh.at[pl.ds(r0, rows_per_tile)])
    plsc.subcore_barrier()

    def gb_of(g, j):
      return (g * NBUF + j) * NW + wid

    for j in range(NBUF):
      pltpu.sync_copy(edge_hbm.at[0, gb_of(0, j)], sidx[j])
      pltpu.async_copy(edge_hbm.at[1, gb_of(0, j)], didx[j], semid)
    for j in range(NBUF):
      pltpu.async_copy(xt_hbm.at[sidx[j]], rows[j], semg[j])

    def body(g, carry):
      for j in range(NBUF):
        pltpu.make_async_copy(edge_hbm.at[1, wid], didx[j], semid).wait()
      for j in range(NBUF):
        pltpu.make_async_copy(xt_hbm.at[sidx[j]], rows[j], semg[j]).wait()
        pltpu.async_copy(rows[j], acc_sh.at[didx[j]], sems[j], add=True)
        pltpu.async_copy(edge_hbm.at[0, gb_of(g + 1, j)], sidx[j], semis)
      for j in range(NBUF):
        pltpu.make_async_copy(edge_hbm.at[0, wid], sidx[j], semis).wait()
      for j in range(NBUF):
        pltpu.make_async_copy(rows[j], acc_sh.at[didx[j]], sems[j]).wait()
        pltpu.async_copy(xt_hbm.at[sidx[j]], rows[j], semg[j])
        pltpu.async_copy(edge_hbm.at[1, gb_of(g + 1, j)], didx[j], semid)
      return carry

    lax.fori_loop(0, ngrp - 1, body, 0)

    for j in range(NBUF):
      pltpu.make_async_copy(edge_hbm.at[1, wid], didx[j], semid).wait()
    for j in range(NBUF):
      pltpu.make_async_copy(xt_hbm.at[sidx[j]], rows[j], semg[j]).wait()
      pltpu.async_copy(rows[j], acc_sh.at[didx[j]], sems[j], add=True)
    for j in range(NBUF):
      pltpu.make_async_copy(rows[j], acc_sh.at[didx[j]], sems[j]).wait()
    plsc.subcore_barrier()

    pltpu.sync_copy(acc_sh.at[pl.ds(r0, rows_per_tile)],
                    out_hbm.at[pl.ds(c * npad + r0, rows_per_tile)])

  return agg_kernel


def _make_scale_body(n):
  def _scale_body(x_ref, dd_ref, o_ref):
    dd = dd_ref[...]
    cnt = dd[0, :] + dd[1, :] + 1.0
    dis = jnp.expand_dims(lax.rsqrt(cnt), 1)[:n]
    o_ref[...] = x_ref[...] * dis
  return _scale_body


def _make_combine_body(n):
  def _combine_body(ya_ref, yb_ref, x_ref, dd_ref,
                    wc_ref, wl_ref, bc_ref, bl_ref, o_ref):
    dd = dd_ref[...]
    cnt = dd[0, :] + dd[1, :] + 1.0
    dis = jnp.expand_dims(lax.rsqrt(cnt), 1)[:n]
    y = dis * (ya_ref[0, :n] + yb_ref[0, :n]) + (dis * dis) * x_ref[...]
    dn = (((1,), (1,)), ((), ()))
    agg = lax.dot_general(y, wc_ref[...], dn,
                          preferred_element_type=jnp.float32) + bc_ref[...]
    o_ref[...] = lax.dot_general(agg, wl_ref[...], dn,
                                 preferred_element_type=jnp.float32)
    o_ref[...] += bl_ref[...]
  return _combine_body


def kernel(x, edge_index, W_conv, b_conv, W_lin, b_lin):
  n, d = x.shape
  e = edge_index.shape[1]

  npad = ((n + 127) // 128) * 128
  unit = NBUF
  bpw = ((e + NW * EB - 1) // (NW * EB) + unit - 1) // unit * unit
  bpw_deg = (bpw // (2 * DCH)) * (2 * DCH)
  if bpw_deg * NW * EB < e:
    bpw_deg += 2 * DCH
    bpw = max(bpw, ((bpw_deg + unit - 1) // unit) * unit)
  e2 = bpw * NW * EB
  nbatch = e2 // EB

  pad_r = jnp.arange(e2 - e, dtype=jnp.int32)
  pads = jnp.stack([pad_r % n, n + pad_r % (npad - n)])
  edge3 = jnp.concatenate([edge_index, pads], axis=1).reshape(2, nbatch, EB)
  zeros_1d = jnp.zeros((npad,), jnp.float32)
  zeros_2d = jnp.zeros((npad, d), jnp.float32)

  degp = _make_degree_kernel(npad, bpw_deg)(edge3, zeros_1d)
  deg2 = degp.reshape(2, npad)

  dspec = pl.BlockSpec((2, npad), lambda i: (0, 0))
  fspec = pl.BlockSpec((n, d), lambda i: (0, 0))
  xt = pl.pallas_call(
      _make_scale_body(n),
      grid=(1,),
      in_specs=[fspec, dspec],
      out_specs=fspec,
      out_shape=jax.ShapeDtypeStruct((n, d), jnp.float32),
  )(x, deg2)

  y0 = _make_agg_kernel(npad, d, bpw)(xt, edge3, zeros_2d)
  y3 = y0.reshape(2, npad, d)

  h = W_conv.shape[0]
  pspec = pl.BlockSpec((1, npad, d), lambda i: (0, 0, 0))
  out = pl.pallas_call(
      _make_combine_body(n),
      grid=(1,),
      in_specs=[
          pspec,
          pl.BlockSpec((1, npad, d), lambda i: (1, 0, 0)),
          fspec,
          dspec,
          pl.BlockSpec((h, d), lambda i: (0, 0)),
          pl.BlockSpec((d, h), lambda i: (0, 0)),
          pl.BlockSpec((1, h), lambda i: (0, 0)),
          pl.BlockSpec((1, d), lambda i: (0, 0)),
      ],
      out_specs=fspec,
      out_shape=jax.ShapeDtypeStruct((n, d), jnp.float32),
  )(y3, y3, x, deg2,
    W_conv, W_lin, b_conv.reshape(1, h), b_lin.reshape(1, d))

  return out
